# Initial kernel scaffold; baseline (speedup 1.0000x reference)
import jax
import jax.numpy as jnp
from jax.experimental import pallas as pl


def kernel(x, edge_index, edge_weight, input_mask, W_src, att_src, att_dst, W_edge, att_edge, bias_gat, W_B, b_B, W_W, b_W):
    raise NotImplementedError("write your pallas kernel here")


# speedup vs baseline: 18.3425x; 18.3425x over previous
import functools

import jax
import jax.numpy as jnp
from jax import lax
from jax.experimental import pallas as pl
from jax.experimental.pallas import tpu as pltpu
from jax.experimental.pallas import tpu_sc as plsc

NC = 2
NS = 16
NW = NC * NS

N_PAD = 50176
ROWS_PER_TILE = N_PAD // NS
E_PAD = 819200
EDGES_PER_TILE = E_PAD // NW
KB = 128
CHUNKS = EDGES_PER_TILE // KB
KD = 1024
CHUNKS_D = EDGES_PER_TILE // KD
BLK = 512


def _mm_body(x_ref, w_ref, o_ref):
    o_ref[...] = lax.dot_general(
        x_ref[...], w_ref[...], (((1,), (0,)), ((), ())),
        precision=lax.Precision.HIGHEST)


def _project(x_pad, w_aug):
    return pl.pallas_call(
        _mm_body,
        grid=(N_PAD // BLK,),
        in_specs=[
            pl.BlockSpec((BLK, 512), lambda i: (i, 0)),
            pl.BlockSpec((512, 128), lambda i: (0, 0)),
        ],
        out_specs=pl.BlockSpec((BLK, 128), lambda i: (i, 0)),
        out_shape=jax.ShapeDtypeStruct((N_PAD, 128), jnp.float32),
    )(x_pad, w_aug)


def _edge_body(src_hbm, dst_hbm, ew_hbm, asrc_hbm, adst_hbm, h_hbm,
               out_hbm, den_hbm, deg_hbm, sw_hbm,
               hrows, src_c, dst_c, ew_c, ex_c, ones_c, av_c, bv_c,
               out_sh, den_sh, deg_sh, sw_sh, sem, sem2, sem3):
    c = lax.axis_index("c")
    s = lax.axis_index("s")

    zero16 = jnp.zeros((16,), jnp.float32)
    one16 = jnp.ones((16,), jnp.float32)

    def _zrow(j, carry):
        hrows[j, 0:16] = zero16
        hrows[j, 16:32] = zero16
        return carry
    lax.fori_loop(0, KB, _zrow, 0)

    def _zbuf(g, carry):
        i16 = pl.ds(g * 16, 16)
        ex_c[i16] = zero16
        ones_c[i16] = one16
        return carry
    lax.fori_loop(0, KB // 16, _zbuf, 0)

    r0 = s * ROWS_PER_TILE
    n_full = ROWS_PER_TILE // KB
    for k in range(ROWS_PER_TILE // KB):
        pltpu.sync_copy(hrows, out_sh.at[pl.ds(r0 + k * KB, KB)])
        pltpu.sync_copy(ex_c, den_sh.at[pl.ds(r0 + k * KB, KB)])
        pltpu.sync_copy(ex_c, deg_sh.at[pl.ds(r0 + k * KB, KB)])
        pltpu.sync_copy(ex_c, sw_sh.at[pl.ds(r0 + k * KB, KB)])
    rem = ROWS_PER_TILE - (ROWS_PER_TILE // KB) * KB
    if rem:
        rb = r0 + (ROWS_PER_TILE // KB) * KB
        pltpu.sync_copy(hrows.at[pl.ds(0, rem)], out_sh.at[pl.ds(rb, rem)])
        pltpu.sync_copy(ex_c.at[pl.ds(0, rem)], den_sh.at[pl.ds(rb, rem)])
        pltpu.sync_copy(ex_c.at[pl.ds(0, rem)], deg_sh.at[pl.ds(rb, rem)])
        pltpu.sync_copy(ex_c.at[pl.ds(0, rem)], sw_sh.at[pl.ds(rb, rem)])

    plsc.subcore_barrier()

    base = (c * NS + s) * EDGES_PER_TILE

    def _chunk(t, carry):
        e0 = base + t * KB
        pltpu.sync_copy(src_hbm.at[pl.ds(e0, KB)], src_c)
        pltpu.sync_copy(dst_hbm.at[pl.ds(e0, KB)], dst_c)
        pltpu.sync_copy(ew_hbm.at[pl.ds(e0, KB)], ew_c)
        cp = pltpu.async_copy(h_hbm.at[src_c], hrows, sem)
        cpa = pltpu.async_copy(asrc_hbm.at[src_c], av_c, sem2)
        cpb = pltpu.async_copy(adst_hbm.at[dst_c], bv_c, sem3)
        cpa.wait()
        cpb.wait()

        def _grp(g, cc):
            i16 = pl.ds(g * 16, 16)
            al = av_c[i16] + bv_c[i16] + ew_c[i16]
            al = jnp.where(al >= 0.0, al, 0.2 * al)
            ex_c[i16] = jnp.exp(al)
            return cc
        lax.fori_loop(0, KB // 16, _grp, 0)

        cp.wait()

        def _scale(j, cc):
            sv = plsc.load_gather(ex_c, [jnp.full((16,), j, jnp.int32)])
            hrows[j, 0:16] = hrows[j, 0:16] * sv
            hrows[j, 16:32] = hrows[j, 16:32] * sv
            return cc
        lax.fori_loop(0, KB, _scale, 0)

        pltpu.sync_copy(hrows, out_sh.at[dst_c], add=True)
        pltpu.sync_copy(ex_c, den_sh.at[dst_c], add=True)
        pltpu.sync_copy(ones_c, deg_sh.at[dst_c], add=True)
        pltpu.sync_copy(ew_c, sw_sh.at[dst_c], add=True)
        return carry

    lax.fori_loop(0, CHUNKS, _chunk, 0)

    plsc.subcore_barrier()

    w0 = c * N_PAD + r0
    pltpu.sync_copy(out_sh.at[pl.ds(r0, ROWS_PER_TILE)],
                    out_hbm.at[pl.ds(w0, ROWS_PER_TILE)])
    pltpu.sync_copy(den_sh.at[pl.ds(r0, ROWS_PER_TILE)],
                    den_hbm.at[pl.ds(w0, ROWS_PER_TILE)])
    pltpu.sync_copy(deg_sh.at[pl.ds(r0, ROWS_PER_TILE)],
                    deg_hbm.at[pl.ds(w0, ROWS_PER_TILE)])
    pltpu.sync_copy(sw_sh.at[pl.ds(r0, ROWS_PER_TILE)],
                    sw_hbm.at[pl.ds(w0, ROWS_PER_TILE)])


def _edge_pass(srcp, dstp, ewp, asrc, adst, h):
    mesh = plsc.VectorSubcoreMesh(
        core_axis_name="c", subcore_axis_name="s",
        num_cores=NC, num_subcores=NS)
    return pl.kernel(
        _edge_body,
        out_type=[
            jax.ShapeDtypeStruct((NC * N_PAD, 32), jnp.float32),
            jax.ShapeDtypeStruct((NC * N_PAD,), jnp.float32),
            jax.ShapeDtypeStruct((NC * N_PAD,), jnp.float32),
            jax.ShapeDtypeStruct((NC * N_PAD,), jnp.float32),
        ],
        mesh=mesh,
        scratch_types=[
            pltpu.VMEM((KB, 32), jnp.float32),
            pltpu.VMEM((KB,), jnp.int32),
            pltpu.VMEM((KB,), jnp.int32),
            pltpu.VMEM((KB,), jnp.float32),
            pltpu.VMEM((KB,), jnp.float32),
            pltpu.VMEM((KB,), jnp.float32),
            pltpu.VMEM((KB,), jnp.float32),
            pltpu.VMEM((KB,), jnp.float32),
            pltpu.VMEM_SHARED((N_PAD, 32), jnp.float32),
            pltpu.VMEM_SHARED((N_PAD,), jnp.float32),
            pltpu.VMEM_SHARED((N_PAD,), jnp.float32),
            pltpu.VMEM_SHARED((N_PAD,), jnp.float32),
            pltpu.SemaphoreType.DMA,
            pltpu.SemaphoreType.DMA,
            pltpu.SemaphoreType.DMA,
        ],
        compiler_params=pltpu.CompilerParams(needs_layout_passes=False, use_tc_tiling_on_sc=False),
    )(srcp, dstp, ewp, asrc, adst, h)


def _epilogue_body(op_ref, s_ref, h_ref, m_ref, v_ref, b_ref, y_ref):
    S = s_ref[...]
    den = S[:, 0:1] + S[:, 1:2]
    deg = S[:, 2:3] + S[:, 3:4]
    sw = S[:, 4:5] + S[:, 5:6]
    al = S[:, 6:7] + S[:, 7:8] + sw / jnp.maximum(deg, 1.0)
    al = jnp.where(al >= 0.0, al, 0.2 * al)
    exs = jnp.exp(al)
    h = h_ref[...]
    out = (op_ref[0] + op_ref[1] + exs * h) / (den + exs + 1e-16)
    xr = jnp.maximum(out + v_ref[0:1, :], 0.0)
    bias = jnp.sum(xr * v_ref[1:2, :], axis=1, keepdims=True) + v_ref[3:4, 0:1]
    b_ref[...] = bias * m_ref[...]
    y_ref[...] = (jnp.sum(xr * v_ref[2:3, :], axis=1, keepdims=True)
                  + v_ref[3:4, 1:2])


def _epilogue(out_p, scal, h, maskp, vecs):
    nb = N_PAD // BLK
    return pl.pallas_call(
        _epilogue_body,
        grid=(nb,),
        in_specs=[
            pl.BlockSpec((2, BLK, 32), lambda i: (0, i, 0)),
            pl.BlockSpec((BLK, 8), lambda i: (i, 0)),
            pl.BlockSpec((BLK, 32), lambda i: (i, 0)),
            pl.BlockSpec((BLK, 1), lambda i: (i, 0)),
            pl.BlockSpec((4, 32), lambda i: (0, 0)),
        ],
        out_specs=[
            pl.BlockSpec((BLK, 1), lambda i: (i, 0)),
            pl.BlockSpec((BLK, 1), lambda i: (i, 0)),
        ],
        out_shape=[
            jax.ShapeDtypeStruct((N_PAD, 1), jnp.float32),
            jax.ShapeDtypeStruct((N_PAD, 1), jnp.float32),
        ],
    )(out_p, scal, h, maskp, vecs)


def _weights_body(src_hbm, dst_hbm, yp_hbm, w_hbm, yp_v, src_c, dst_c, w_c):
    c = lax.axis_index("c")
    s = lax.axis_index("s")
    pltpu.sync_copy(yp_hbm, yp_v)
    base = (c * NS + s) * EDGES_PER_TILE

    def _chunk(t, carry):
        e0 = base + t * KD
        pltpu.sync_copy(src_hbm.at[pl.ds(e0, KD)], src_c)
        pltpu.sync_copy(dst_hbm.at[pl.ds(e0, KD)], dst_c)

        def _grp(g, cc):
            i16 = pl.ds(g * 16, 16)
            a = plsc.load_gather(yp_v, [src_c[i16]])
            b = plsc.load_gather(yp_v, [dst_c[i16]])
            w_c[i16] = 0.5 * (a + b)
            return cc
        lax.fori_loop(0, KD // 16, _grp, 0)

        pltpu.sync_copy(w_c, w_hbm.at[pl.ds(e0, KD)])
        return carry

    lax.fori_loop(0, CHUNKS_D, _chunk, 0)


def _edge_weights(srcp, dstp, yp):
    mesh = plsc.VectorSubcoreMesh(
        core_axis_name="c", subcore_axis_name="s",
        num_cores=NC, num_subcores=NS)
    return pl.kernel(
        _weights_body,
        out_type=jax.ShapeDtypeStruct((E_PAD,), jnp.float32),
        mesh=mesh,
        scratch_types=[
            pltpu.VMEM((N_PAD,), jnp.float32),
            pltpu.VMEM((KD,), jnp.int32),
            pltpu.VMEM((KD,), jnp.int32),
            pltpu.VMEM((KD,), jnp.float32),
        ],
        compiler_params=pltpu.CompilerParams(needs_layout_passes=False, use_tc_tiling_on_sc=False),
    )(srcp, dstp, yp)


def kernel(x, edge_index, edge_weight, input_mask, W_src, att_src, att_dst,
           W_edge, att_edge, bias_gat, W_B, b_B, W_W, b_W):
    n, d_in = x.shape
    e = edge_index.shape[1]

    src = edge_index[0]
    dst = edge_index[1]

    pad_e = E_PAD - e
    pad_idx = jnp.full((pad_e,), n, jnp.int32)
    srcp = jnp.concatenate([src, pad_idx])
    dstp = jnp.concatenate([dst, pad_idx])

    c1 = jnp.dot(W_edge[0], att_edge)
    ewp = jnp.concatenate([edge_weight[:, 0] * c1,
                           jnp.zeros((pad_e,), jnp.float32)])

    v_src = W_src @ att_src
    v_dst = W_src @ att_dst
    w_aug = jnp.concatenate(
        [W_src, v_src[:, None], v_dst[:, None]], axis=1)
    w_aug = jnp.pad(w_aug, ((0, 512 - d_in), (0, 128 - 34)))
    x_pad = jnp.pad(x, ((0, N_PAD - n), (0, 512 - d_in)))

    h_aug = _project(x_pad, w_aug)
    h = h_aug[:, :32]
    asrc = h_aug[:, 32]
    adst = h_aug[:, 33]

    out_f, den_f, deg_f, sw_f = _edge_pass(srcp, dstp, ewp, asrc, adst, h)
    out_p = out_f.reshape(NC, N_PAD, 32)
    den_p = den_f.reshape(NC, N_PAD)
    deg_p = deg_f.reshape(NC, N_PAD)
    sw_p = sw_f.reshape(NC, N_PAD)

    scal = jnp.stack(
        [den_p[0], den_p[1], deg_p[0], deg_p[1], sw_p[0], sw_p[1],
         asrc, adst], axis=1)
    maskp = jnp.pad(input_mask, ((0, N_PAD - n), (0, 0)))
    vecs = jnp.stack(
        [bias_gat, W_B[:, 0], W_W[:, 0],
         jnp.concatenate([b_B, b_W, jnp.zeros((30,), jnp.float32)])])

    biases_full, yp_full = _epilogue(out_p, scal, h, maskp, vecs)

    w_pad = _edge_weights(srcp, dstp, yp_full[:, 0])

    return (w_pad[:e, None], biases_full[:n])


# speedup vs baseline: 23.1944x; 1.2645x over previous
import functools

import jax
import jax.numpy as jnp
from jax import lax
from jax.experimental import pallas as pl
from jax.experimental.pallas import tpu as pltpu
from jax.experimental.pallas import tpu_sc as plsc

NC = 2
NS = 16
NW = NC * NS

N_PAD = 50176
ROWS_PER_TILE = N_PAD // NS
E_PAD = 819200
EDGES_PER_TILE = E_PAD // NW
KB = 128
CHUNKS = EDGES_PER_TILE // KB
KD = 1024
CHUNKS_D = EDGES_PER_TILE // KD
BLK = 512


def _mm_body(x_ref, w_ref, o_ref):
    o_ref[...] = lax.dot_general(
        x_ref[...], w_ref[...], (((1,), (0,)), ((), ())),
        precision=lax.Precision.HIGHEST)


def _project(x, w_aug):
    n, d_in = x.shape
    blk = 400
    return pl.pallas_call(
        _mm_body,
        grid=(n // blk,),
        in_specs=[
            pl.BlockSpec((blk, d_in), lambda i: (i, 0)),
            pl.BlockSpec((d_in, 128), lambda i: (0, 0)),
        ],
        out_specs=pl.BlockSpec((blk, 128), lambda i: (i, 0)),
        out_shape=jax.ShapeDtypeStruct((n, 128), jnp.float32),
    )(x, w_aug)


def _edge_body(src_hbm, dst_hbm, ew_hbm, asrc_hbm, adst_hbm, h_hbm,
               out_hbm, den_hbm, deg_hbm, sw_hbm,
               hrows, src_c, dst_c, ew_c, ex_c, ones_c, av_c, bv_c,
               out_sh, den_sh, deg_sh, sw_sh, sem, sem2, sem3):
    c = lax.axis_index("c")
    s = lax.axis_index("s")

    zero16 = jnp.zeros((16,), jnp.float32)
    one16 = jnp.ones((16,), jnp.float32)

    def _zrow(j, carry):
        hrows[j, 0:16] = zero16
        hrows[j, 16:32] = zero16
        return carry
    lax.fori_loop(0, KB, _zrow, 0)

    def _zbuf(g, carry):
        i16 = pl.ds(g * 16, 16)
        ex_c[i16] = zero16
        ones_c[i16] = one16
        return carry
    lax.fori_loop(0, KB // 16, _zbuf, 0)

    r0 = s * ROWS_PER_TILE
    n_full = ROWS_PER_TILE // KB
    for k in range(ROWS_PER_TILE // KB):
        pltpu.sync_copy(hrows, out_sh.at[pl.ds(r0 + k * KB, KB)])
        pltpu.sync_copy(ex_c, den_sh.at[pl.ds(r0 + k * KB, KB)])
        pltpu.sync_copy(ex_c, deg_sh.at[pl.ds(r0 + k * KB, KB)])
        pltpu.sync_copy(ex_c, sw_sh.at[pl.ds(r0 + k * KB, KB)])
    rem = ROWS_PER_TILE - (ROWS_PER_TILE // KB) * KB
    if rem:
        rb = r0 + (ROWS_PER_TILE // KB) * KB
        pltpu.sync_copy(hrows.at[pl.ds(0, rem)], out_sh.at[pl.ds(rb, rem)])
        pltpu.sync_copy(ex_c.at[pl.ds(0, rem)], den_sh.at[pl.ds(rb, rem)])
        pltpu.sync_copy(ex_c.at[pl.ds(0, rem)], deg_sh.at[pl.ds(rb, rem)])
        pltpu.sync_copy(ex_c.at[pl.ds(0, rem)], sw_sh.at[pl.ds(rb, rem)])

    plsc.subcore_barrier()

    base = (c * NS + s) * EDGES_PER_TILE

    def _chunk(t, carry):
        e0 = base + t * KB
        pltpu.sync_copy(src_hbm.at[pl.ds(e0, KB)], src_c)
        pltpu.sync_copy(dst_hbm.at[pl.ds(e0, KB)], dst_c)
        pltpu.sync_copy(ew_hbm.at[pl.ds(e0, KB)], ew_c)
        cp = pltpu.async_copy(h_hbm.at[src_c], hrows, sem)
        cpa = pltpu.async_copy(asrc_hbm.at[src_c], av_c, sem2)
        cpb = pltpu.async_copy(adst_hbm.at[dst_c], bv_c, sem3)
        cpa.wait()
        cpb.wait()

        def _grp(g, cc):
            i16 = pl.ds(g * 16, 16)
            al = av_c[i16] + bv_c[i16] + ew_c[i16]
            al = jnp.where(al >= 0.0, al, 0.2 * al)
            ex_c[i16] = jnp.exp(al)
            return cc
        lax.fori_loop(0, KB // 16, _grp, 0)

        cp.wait()

        def _scale(j, cc):
            sv = plsc.load_gather(ex_c, [jnp.full((16,), j, jnp.int32)])
            hrows[j, 0:16] = hrows[j, 0:16] * sv
            hrows[j, 16:32] = hrows[j, 16:32] * sv
            return cc
        lax.fori_loop(0, KB, _scale, 0)

        pltpu.sync_copy(hrows, out_sh.at[dst_c], add=True)
        pltpu.sync_copy(ex_c, den_sh.at[dst_c], add=True)
        pltpu.sync_copy(ones_c, deg_sh.at[dst_c], add=True)
        pltpu.sync_copy(ew_c, sw_sh.at[dst_c], add=True)
        return carry

    lax.fori_loop(0, CHUNKS, _chunk, 0)

    plsc.subcore_barrier()

    w0 = c * N_PAD + r0
    pltpu.sync_copy(out_sh.at[pl.ds(r0, ROWS_PER_TILE)],
                    out_hbm.at[pl.ds(w0, ROWS_PER_TILE)])
    pltpu.sync_copy(den_sh.at[pl.ds(r0, ROWS_PER_TILE)],
                    den_hbm.at[pl.ds(w0, ROWS_PER_TILE)])
    pltpu.sync_copy(deg_sh.at[pl.ds(r0, ROWS_PER_TILE)],
                    deg_hbm.at[pl.ds(w0, ROWS_PER_TILE)])
    pltpu.sync_copy(sw_sh.at[pl.ds(r0, ROWS_PER_TILE)],
                    sw_hbm.at[pl.ds(w0, ROWS_PER_TILE)])


def _edge_pass(srcp, dstp, ewp, asrc, adst, h):
    mesh = plsc.VectorSubcoreMesh(
        core_axis_name="c", subcore_axis_name="s",
        num_cores=NC, num_subcores=NS)
    return pl.kernel(
        _edge_body,
        out_type=[
            jax.ShapeDtypeStruct((NC * N_PAD, 32), jnp.float32),
            jax.ShapeDtypeStruct((NC * N_PAD,), jnp.float32),
            jax.ShapeDtypeStruct((NC * N_PAD,), jnp.float32),
            jax.ShapeDtypeStruct((NC * N_PAD,), jnp.float32),
        ],
        mesh=mesh,
        scratch_types=[
            pltpu.VMEM((KB, 32), jnp.float32),
            pltpu.VMEM((KB,), jnp.int32),
            pltpu.VMEM((KB,), jnp.int32),
            pltpu.VMEM((KB,), jnp.float32),
            pltpu.VMEM((KB,), jnp.float32),
            pltpu.VMEM((KB,), jnp.float32),
            pltpu.VMEM((KB,), jnp.float32),
            pltpu.VMEM((KB,), jnp.float32),
            pltpu.VMEM_SHARED((N_PAD, 32), jnp.float32),
            pltpu.VMEM_SHARED((N_PAD,), jnp.float32),
            pltpu.VMEM_SHARED((N_PAD,), jnp.float32),
            pltpu.VMEM_SHARED((N_PAD,), jnp.float32),
            pltpu.SemaphoreType.DMA,
            pltpu.SemaphoreType.DMA,
            pltpu.SemaphoreType.DMA,
        ],
        compiler_params=pltpu.CompilerParams(needs_layout_passes=False, use_tc_tiling_on_sc=False),
    )(srcp, dstp, ewp, asrc, adst, h)


def _epilogue_body(op_ref, s_ref, h_ref, m_ref, v_ref, b_ref, y_ref):
    S = s_ref[...]
    den = S[:, 0:1] + S[:, 1:2]
    deg = S[:, 2:3] + S[:, 3:4]
    sw = S[:, 4:5] + S[:, 5:6]
    al = S[:, 6:7] + S[:, 7:8] + sw / jnp.maximum(deg, 1.0)
    al = jnp.where(al >= 0.0, al, 0.2 * al)
    exs = jnp.exp(al)
    h = h_ref[...]
    out = (op_ref[0] + op_ref[1] + exs * h) / (den + exs + 1e-16)
    xr = jnp.maximum(out + v_ref[0:1, :], 0.0)
    bias = jnp.sum(xr * v_ref[1:2, :], axis=1, keepdims=True) + v_ref[3:4, 0:1]
    b_ref[...] = bias * m_ref[...]
    y_ref[...] = (jnp.sum(xr * v_ref[2:3, :], axis=1, keepdims=True)
                  + v_ref[3:4, 1:2])


def _epilogue(out_p, scal, h, maskp, vecs):
    nb = N_PAD // BLK
    return pl.pallas_call(
        _epilogue_body,
        grid=(nb,),
        in_specs=[
            pl.BlockSpec((2, BLK, 32), lambda i: (0, i, 0)),
            pl.BlockSpec((BLK, 8), lambda i: (i, 0)),
            pl.BlockSpec((BLK, 32), lambda i: (i, 0)),
            pl.BlockSpec((BLK, 1), lambda i: (i, 0)),
            pl.BlockSpec((4, 32), lambda i: (0, 0)),
        ],
        out_specs=[
            pl.BlockSpec((BLK, 1), lambda i: (i, 0)),
            pl.BlockSpec((BLK, 1), lambda i: (i, 0)),
        ],
        out_shape=[
            jax.ShapeDtypeStruct((N_PAD, 1), jnp.float32),
            jax.ShapeDtypeStruct((N_PAD, 1), jnp.float32),
        ],
    )(out_p, scal, h, maskp, vecs)


def _weights_body(src_hbm, dst_hbm, yp_hbm, w_hbm, yp_v, src_c, dst_c, w_c):
    c = lax.axis_index("c")
    s = lax.axis_index("s")
    pltpu.sync_copy(yp_hbm, yp_v)
    base = (c * NS + s) * EDGES_PER_TILE

    def _chunk(t, carry):
        e0 = base + t * KD
        pltpu.sync_copy(src_hbm.at[pl.ds(e0, KD)], src_c)
        pltpu.sync_copy(dst_hbm.at[pl.ds(e0, KD)], dst_c)

        def _grp(g, cc):
            i16 = pl.ds(g * 16, 16)
            a = plsc.load_gather(yp_v, [src_c[i16]])
            b = plsc.load_gather(yp_v, [dst_c[i16]])
            w_c[i16] = 0.5 * (a + b)
            return cc
        lax.fori_loop(0, KD // 16, _grp, 0)

        pltpu.sync_copy(w_c, w_hbm.at[pl.ds(e0, KD)])
        return carry

    lax.fori_loop(0, CHUNKS_D, _chunk, 0)


def _edge_weights(srcp, dstp, yp):
    mesh = plsc.VectorSubcoreMesh(
        core_axis_name="c", subcore_axis_name="s",
        num_cores=NC, num_subcores=NS)
    return pl.kernel(
        _weights_body,
        out_type=jax.ShapeDtypeStruct((E_PAD,), jnp.float32),
        mesh=mesh,
        scratch_types=[
            pltpu.VMEM((N_PAD,), jnp.float32),
            pltpu.VMEM((KD,), jnp.int32),
            pltpu.VMEM((KD,), jnp.int32),
            pltpu.VMEM((KD,), jnp.float32),
        ],
        compiler_params=pltpu.CompilerParams(needs_layout_passes=False, use_tc_tiling_on_sc=False),
    )(srcp, dstp, yp)


def kernel(x, edge_index, edge_weight, input_mask, W_src, att_src, att_dst,
           W_edge, att_edge, bias_gat, W_B, b_B, W_W, b_W):
    n, d_in = x.shape
    e = edge_index.shape[1]

    src = edge_index[0]
    dst = edge_index[1]

    pad_e = E_PAD - e
    pad_idx = jnp.full((pad_e,), n, jnp.int32)
    srcp = jnp.concatenate([src, pad_idx])
    dstp = jnp.concatenate([dst, pad_idx])

    c1 = jnp.dot(W_edge[0], att_edge)
    ewp = jnp.concatenate([edge_weight[:, 0] * c1,
                           jnp.zeros((pad_e,), jnp.float32)])

    v_src = W_src @ att_src
    v_dst = W_src @ att_dst
    w_aug = jnp.concatenate(
        [W_src, v_src[:, None], v_dst[:, None]], axis=1)
    w_aug = jnp.pad(w_aug, ((0, 0), (0, 128 - 34)))

    h_aug = _project(x, w_aug)
    h = jnp.pad(h_aug[:, :32], ((0, N_PAD - n), (0, 0)))
    asrc = jnp.pad(h_aug[:, 32], (0, N_PAD - n))
    adst = jnp.pad(h_aug[:, 33], (0, N_PAD - n))

    out_f, den_f, deg_f, sw_f = _edge_pass(srcp, dstp, ewp, asrc, adst, h)
    out_p = out_f.reshape(NC, N_PAD, 32)
    den_p = den_f.reshape(NC, N_PAD)
    deg_p = deg_f.reshape(NC, N_PAD)
    sw_p = sw_f.reshape(NC, N_PAD)

    scal = jnp.stack(
        [den_p[0], den_p[1], deg_p[0], deg_p[1], sw_p[0], sw_p[1],
         asrc, adst], axis=1)
    maskp = jnp.pad(input_mask, ((0, N_PAD - n), (0, 0)))
    vecs = jnp.stack(
        [bias_gat, W_B[:, 0], W_W[:, 0],
         jnp.concatenate([b_B, b_W, jnp.zeros((30,), jnp.float32)])])

    biases_full, yp_full = _epilogue(out_p, scal, h, maskp, vecs)

    w_pad = _edge_weights(srcp, dstp, yp_full[:, 0])

    return (w_pad[:e, None], biases_full[:n])


# speedup vs baseline: 29.2131x; 1.2595x over previous
import functools

import jax
import jax.numpy as jnp
from jax import lax
from jax.experimental import pallas as pl
from jax.experimental.pallas import tpu as pltpu
from jax.experimental.pallas import tpu_sc as plsc

NC = 2
NS = 16
NW = NC * NS

N_PAD = 50176
ROWS_PER_TILE = N_PAD // NS
E_PAD = 819200
EDGES_PER_TILE = E_PAD // NW
KB = 128
CHUNKS = EDGES_PER_TILE // KB
KD = 1024
CHUNKS_D = EDGES_PER_TILE // KD
BLK = 512


def _mm_body(x_ref, w_ref, o_ref):
    o_ref[...] = lax.dot_general(
        x_ref[...], w_ref[...], (((1,), (0,)), ((), ())),
        precision=lax.Precision.HIGHEST)


def _project(x, w_aug):
    n, d_in = x.shape
    blk = 400
    return pl.pallas_call(
        _mm_body,
        grid=(n // blk,),
        in_specs=[
            pl.BlockSpec((blk, d_in), lambda i: (i, 0)),
            pl.BlockSpec((d_in, 128), lambda i: (0, 0)),
        ],
        out_specs=pl.BlockSpec((blk, 128), lambda i: (i, 0)),
        out_shape=jax.ShapeDtypeStruct((n, 128), jnp.float32),
    )(x, w_aug)


def _edge_body(src_hbm, dst_hbm, ew_hbm, asrc_hbm, adst_hbm, h_hbm,
               out_hbm, den_hbm, deg_hbm, sw_hbm,
               hrows, src_c, dst_c, ew_c, ex_c, ones_c, av_c, bv_c,
               hrows2, src_c2, dst_c2, ew_c2, ex_c2, av_c2, bv_c2,
               out_sh, den_sh, deg_sh, sw_sh,
               semL0, semL1, semG0, semG1):
    c = lax.axis_index("c")
    s = lax.axis_index("s")

    zero16 = jnp.zeros((16,), jnp.float32)
    one16 = jnp.ones((16,), jnp.float32)

    def _zrow(j, carry):
        hrows[j, 0:16] = zero16
        hrows[j, 16:32] = zero16
        return carry
    lax.fori_loop(0, KB, _zrow, 0)

    def _zbuf(g, carry):
        i16 = pl.ds(g * 16, 16)
        ex_c[i16] = zero16
        ones_c[i16] = one16
        return carry
    lax.fori_loop(0, KB // 16, _zbuf, 0)

    r0 = s * ROWS_PER_TILE
    n_full = ROWS_PER_TILE // KB
    for k in range(ROWS_PER_TILE // KB):
        pltpu.sync_copy(hrows, out_sh.at[pl.ds(r0 + k * KB, KB)])
        pltpu.sync_copy(ex_c, den_sh.at[pl.ds(r0 + k * KB, KB)])
        pltpu.sync_copy(ex_c, deg_sh.at[pl.ds(r0 + k * KB, KB)])
        pltpu.sync_copy(ex_c, sw_sh.at[pl.ds(r0 + k * KB, KB)])
    rem = ROWS_PER_TILE - (ROWS_PER_TILE // KB) * KB
    if rem:
        rb = r0 + (ROWS_PER_TILE // KB) * KB
        pltpu.sync_copy(hrows.at[pl.ds(0, rem)], out_sh.at[pl.ds(rb, rem)])
        pltpu.sync_copy(ex_c.at[pl.ds(0, rem)], den_sh.at[pl.ds(rb, rem)])
        pltpu.sync_copy(ex_c.at[pl.ds(0, rem)], deg_sh.at[pl.ds(rb, rem)])
        pltpu.sync_copy(ex_c.at[pl.ds(0, rem)], sw_sh.at[pl.ds(rb, rem)])

    plsc.subcore_barrier()

    base = (c * NS + s) * EDGES_PER_TILE
    hrowsB = (hrows, hrows2)
    srcB = (src_c, src_c2)
    dstB = (dst_c, dst_c2)
    ewB = (ew_c, ew_c2)
    exB = (ex_c, ex_c2)
    avB = (av_c, av_c2)
    bvB = (bv_c, bv_c2)
    semLB = (semL0, semL1)
    semGB = (semG0, semG1)

    def _issue_loads(b, t):
        e0 = base + t * KB
        pltpu.async_copy(src_hbm.at[pl.ds(e0, KB)], srcB[b], semLB[b])
        pltpu.async_copy(dst_hbm.at[pl.ds(e0, KB)], dstB[b], semLB[b])
        pltpu.async_copy(ew_hbm.at[pl.ds(e0, KB)], ewB[b], semLB[b])

    def _wait_loads(b):
        pltpu.make_async_copy(src_hbm.at[pl.ds(0, KB)], srcB[b], semLB[b]).wait()
        pltpu.make_async_copy(dst_hbm.at[pl.ds(0, KB)], dstB[b], semLB[b]).wait()
        pltpu.make_async_copy(ew_hbm.at[pl.ds(0, KB)], ewB[b], semLB[b]).wait()

    def _issue_gathers(b):
        pltpu.async_copy(h_hbm.at[srcB[b]], hrowsB[b], semGB[b])
        pltpu.async_copy(asrc_hbm.at[srcB[b]], avB[b], semGB[b])
        pltpu.async_copy(adst_hbm.at[dstB[b]], bvB[b], semGB[b])

    def _wait_gathers(b):
        pltpu.make_async_copy(h_hbm.at[pl.ds(0, KB)], hrowsB[b], semGB[b]).wait()
        pltpu.make_async_copy(asrc_hbm.at[pl.ds(0, KB)], avB[b], semGB[b]).wait()
        pltpu.make_async_copy(adst_hbm.at[pl.ds(0, KB)], bvB[b], semGB[b]).wait()

    _issue_loads(0, 0)
    _wait_loads(0)
    _issue_gathers(0)

    def _macro(i, carry):
        for b in (0, 1):
            t = 2 * i + b
            tn = lax.rem(t + 1, CHUNKS)
            nb = 1 - b
            _issue_loads(nb, tn)

            _wait_gathers(b)

            def _grp(g, cc):
                i16 = pl.ds(g * 16, 16)
                al = avB[b][i16] + bvB[b][i16] + ewB[b][i16]
                al = jnp.where(al >= 0.0, al, 0.2 * al)
                exB[b][i16] = jnp.exp(al)
                return cc
            lax.fori_loop(0, KB // 16, _grp, 0)

            def _scale(j, cc):
                sv = plsc.load_gather(exB[b], [jnp.full((16,), j, jnp.int32)])
                hrowsB[b][j, 0:16] = hrowsB[b][j, 0:16] * sv
                hrowsB[b][j, 16:32] = hrowsB[b][j, 16:32] * sv
                return cc
            lax.fori_loop(0, KB, _scale, 0)

            _wait_loads(nb)
            _issue_gathers(nb)

            pltpu.sync_copy(hrowsB[b], out_sh.at[dstB[b]], add=True)
            pltpu.sync_copy(exB[b], den_sh.at[dstB[b]], add=True)
            pltpu.sync_copy(ones_c, deg_sh.at[dstB[b]], add=True)
            pltpu.sync_copy(ewB[b], sw_sh.at[dstB[b]], add=True)
        return carry

    lax.fori_loop(0, CHUNKS // 2, _macro, 0)

    _wait_gathers(0)

    plsc.subcore_barrier()

    w0 = c * N_PAD + r0
    pltpu.sync_copy(out_sh.at[pl.ds(r0, ROWS_PER_TILE)],
                    out_hbm.at[pl.ds(w0, ROWS_PER_TILE)])
    pltpu.sync_copy(den_sh.at[pl.ds(r0, ROWS_PER_TILE)],
                    den_hbm.at[pl.ds(w0, ROWS_PER_TILE)])
    pltpu.sync_copy(deg_sh.at[pl.ds(r0, ROWS_PER_TILE)],
                    deg_hbm.at[pl.ds(w0, ROWS_PER_TILE)])
    pltpu.sync_copy(sw_sh.at[pl.ds(r0, ROWS_PER_TILE)],
                    sw_hbm.at[pl.ds(w0, ROWS_PER_TILE)])


def _edge_pass(srcp, dstp, ewp, asrc, adst, h):
    mesh = plsc.VectorSubcoreMesh(
        core_axis_name="c", subcore_axis_name="s",
        num_cores=NC, num_subcores=NS)
    return pl.kernel(
        _edge_body,
        out_type=[
            jax.ShapeDtypeStruct((NC * N_PAD, 32), jnp.float32),
            jax.ShapeDtypeStruct((NC * N_PAD,), jnp.float32),
            jax.ShapeDtypeStruct((NC * N_PAD,), jnp.float32),
            jax.ShapeDtypeStruct((NC * N_PAD,), jnp.float32),
        ],
        mesh=mesh,
        scratch_types=[
            pltpu.VMEM((KB, 32), jnp.float32),
            pltpu.VMEM((KB,), jnp.int32),
            pltpu.VMEM((KB,), jnp.int32),
            pltpu.VMEM((KB,), jnp.float32),
            pltpu.VMEM((KB,), jnp.float32),
            pltpu.VMEM((KB,), jnp.float32),
            pltpu.VMEM((KB,), jnp.float32),
            pltpu.VMEM((KB,), jnp.float32),
            pltpu.VMEM((KB, 32), jnp.float32),
            pltpu.VMEM((KB,), jnp.int32),
            pltpu.VMEM((KB,), jnp.int32),
            pltpu.VMEM((KB,), jnp.float32),
            pltpu.VMEM((KB,), jnp.float32),
            pltpu.VMEM((KB,), jnp.float32),
            pltpu.VMEM((KB,), jnp.float32),
            pltpu.VMEM_SHARED((N_PAD, 32), jnp.float32),
            pltpu.VMEM_SHARED((N_PAD,), jnp.float32),
            pltpu.VMEM_SHARED((N_PAD,), jnp.float32),
            pltpu.VMEM_SHARED((N_PAD,), jnp.float32),
            pltpu.SemaphoreType.DMA,
            pltpu.SemaphoreType.DMA,
            pltpu.SemaphoreType.DMA,
            pltpu.SemaphoreType.DMA,
        ],
        compiler_params=pltpu.CompilerParams(needs_layout_passes=False, use_tc_tiling_on_sc=False),
    )(srcp, dstp, ewp, asrc, adst, h)


def _epilogue_body(op_ref, s_ref, h_ref, m_ref, v_ref, b_ref, y_ref):
    S = s_ref[...]
    den = S[:, 0:1] + S[:, 1:2]
    deg = S[:, 2:3] + S[:, 3:4]
    sw = S[:, 4:5] + S[:, 5:6]
    al = S[:, 6:7] + S[:, 7:8] + sw / jnp.maximum(deg, 1.0)
    al = jnp.where(al >= 0.0, al, 0.2 * al)
    exs = jnp.exp(al)
    h = h_ref[...]
    out = (op_ref[0] + op_ref[1] + exs * h) / (den + exs + 1e-16)
    xr = jnp.maximum(out + v_ref[0:1, :], 0.0)
    bias = jnp.sum(xr * v_ref[1:2, :], axis=1, keepdims=True) + v_ref[3:4, 0:1]
    b_ref[...] = bias * m_ref[...]
    y_ref[...] = (jnp.sum(xr * v_ref[2:3, :], axis=1, keepdims=True)
                  + v_ref[3:4, 1:2])


def _epilogue(out_p, scal, h, maskp, vecs):
    nb = N_PAD // BLK
    return pl.pallas_call(
        _epilogue_body,
        grid=(nb,),
        in_specs=[
            pl.BlockSpec((2, BLK, 32), lambda i: (0, i, 0)),
            pl.BlockSpec((BLK, 8), lambda i: (i, 0)),
            pl.BlockSpec((BLK, 32), lambda i: (i, 0)),
            pl.BlockSpec((BLK, 1), lambda i: (i, 0)),
            pl.BlockSpec((4, 32), lambda i: (0, 0)),
        ],
        out_specs=[
            pl.BlockSpec((BLK, 1), lambda i: (i, 0)),
            pl.BlockSpec((BLK, 1), lambda i: (i, 0)),
        ],
        out_shape=[
            jax.ShapeDtypeStruct((N_PAD, 1), jnp.float32),
            jax.ShapeDtypeStruct((N_PAD, 1), jnp.float32),
        ],
    )(out_p, scal, h, maskp, vecs)


def _weights_body(src_hbm, dst_hbm, yp_hbm, w_hbm, yp_v, src_c, dst_c, w_c):
    c = lax.axis_index("c")
    s = lax.axis_index("s")
    pltpu.sync_copy(yp_hbm, yp_v)
    base = (c * NS + s) * EDGES_PER_TILE

    def _chunk(t, carry):
        e0 = base + t * KD
        pltpu.sync_copy(src_hbm.at[pl.ds(e0, KD)], src_c)
        pltpu.sync_copy(dst_hbm.at[pl.ds(e0, KD)], dst_c)

        def _grp(g, cc):
            i16 = pl.ds(g * 16, 16)
            a = plsc.load_gather(yp_v, [src_c[i16]])
            b = plsc.load_gather(yp_v, [dst_c[i16]])
            w_c[i16] = 0.5 * (a + b)
            return cc
        lax.fori_loop(0, KD // 16, _grp, 0)

        pltpu.sync_copy(w_c, w_hbm.at[pl.ds(e0, KD)])
        return carry

    lax.fori_loop(0, CHUNKS_D, _chunk, 0)


def _edge_weights(srcp, dstp, yp):
    mesh = plsc.VectorSubcoreMesh(
        core_axis_name="c", subcore_axis_name="s",
        num_cores=NC, num_subcores=NS)
    return pl.kernel(
        _weights_body,
        out_type=jax.ShapeDtypeStruct((E_PAD,), jnp.float32),
        mesh=mesh,
        scratch_types=[
            pltpu.VMEM((N_PAD,), jnp.float32),
            pltpu.VMEM((KD,), jnp.int32),
            pltpu.VMEM((KD,), jnp.int32),
            pltpu.VMEM((KD,), jnp.float32),
        ],
        compiler_params=pltpu.CompilerParams(needs_layout_passes=False, use_tc_tiling_on_sc=False),
    )(srcp, dstp, yp)


def kernel(x, edge_index, edge_weight, input_mask, W_src, att_src, att_dst,
           W_edge, att_edge, bias_gat, W_B, b_B, W_W, b_W):
    n, d_in = x.shape
    e = edge_index.shape[1]

    src = edge_index[0]
    dst = edge_index[1]

    pad_e = E_PAD - e
    pad_idx = jnp.full((pad_e,), n, jnp.int32)
    srcp = jnp.concatenate([src, pad_idx])
    dstp = jnp.concatenate([dst, pad_idx])

    c1 = jnp.dot(W_edge[0], att_edge)
    ewp = jnp.concatenate([edge_weight[:, 0] * c1,
                           jnp.zeros((pad_e,), jnp.float32)])

    v_src = W_src @ att_src
    v_dst = W_src @ att_dst
    w_aug = jnp.concatenate(
        [W_src, v_src[:, None], v_dst[:, None]], axis=1)
    w_aug = jnp.pad(w_aug, ((0, 0), (0, 128 - 34)))

    h_aug = _project(x, w_aug)
    h = jnp.pad(h_aug[:, :32], ((0, N_PAD - n), (0, 0)))
    asrc = jnp.pad(h_aug[:, 32], (0, N_PAD - n))
    adst = jnp.pad(h_aug[:, 33], (0, N_PAD - n))

    out_f, den_f, deg_f, sw_f = _edge_pass(srcp, dstp, ewp, asrc, adst, h)
    out_p = out_f.reshape(NC, N_PAD, 32)
    den_p = den_f.reshape(NC, N_PAD)
    deg_p = deg_f.reshape(NC, N_PAD)
    sw_p = sw_f.reshape(NC, N_PAD)

    scal = jnp.stack(
        [den_p[0], den_p[1], deg_p[0], deg_p[1], sw_p[0], sw_p[1],
         asrc, adst], axis=1)
    maskp = jnp.pad(input_mask, ((0, N_PAD - n), (0, 0)))
    vecs = jnp.stack(
        [bias_gat, W_B[:, 0], W_W[:, 0],
         jnp.concatenate([b_B, b_W, jnp.zeros((30,), jnp.float32)])])

    biases_full, yp_full = _epilogue(out_p, scal, h, maskp, vecs)

    w_pad = _edge_weights(srcp, dstp, yp_full[:, 0])

    return (w_pad[:e, None], biases_full[:n])


# speedup vs baseline: 29.6854x; 1.0162x over previous
import functools

import jax
import jax.numpy as jnp
from jax import lax
from jax.experimental import pallas as pl
from jax.experimental.pallas import tpu as pltpu
from jax.experimental.pallas import tpu_sc as plsc

NC = 2
NS = 16
NW = NC * NS

N_PAD = 50176
ROWS_PER_TILE = N_PAD // NS
E_PAD = 819200
EDGES_PER_TILE = E_PAD // NW
KB = 128
CHUNKS = EDGES_PER_TILE // KB
KD = 1024
CHUNKS_D = EDGES_PER_TILE // KD
BLK = 512


def _mm_body(x_ref, w_ref, o_ref):
    o_ref[...] = lax.dot_general(
        x_ref[...], w_ref[...], (((1,), (0,)), ((), ())),
        precision=lax.Precision.HIGHEST)


def _project(x, w_aug):
    n, d_in = x.shape
    blk = 400
    return pl.pallas_call(
        _mm_body,
        grid=(n // blk,),
        in_specs=[
            pl.BlockSpec((blk, d_in), lambda i: (i, 0)),
            pl.BlockSpec((d_in, 128), lambda i: (0, 0)),
        ],
        out_specs=pl.BlockSpec((blk, 128), lambda i: (i, 0)),
        out_shape=jax.ShapeDtypeStruct((n, 128), jnp.float32),
    )(x, w_aug)


def _edge_body(src_hbm, dst_hbm, ew_hbm, asrc_hbm, adst_hbm, h_hbm,
               out_hbm, den_hbm, deg_hbm, sw_hbm,
               hrows, src_c, dst_c, ew_c, ex_c, ones_c, av_c, bv_c,
               hrows2, src_c2, dst_c2, ew_c2, ex_c2, av_c2, bv_c2,
               out_sh, den_sh, deg_sh, sw_sh,
               semL0, semL1, semG0, semG1):
    c = lax.axis_index("c")
    s = lax.axis_index("s")

    zero16 = jnp.zeros((16,), jnp.float32)
    one16 = jnp.ones((16,), jnp.float32)

    def _zrow(j, carry):
        hrows[j, 0:16] = zero16
        hrows[j, 16:32] = zero16
        return carry
    lax.fori_loop(0, KB, _zrow, 0)

    def _zbuf(g, carry):
        i16 = pl.ds(g * 16, 16)
        ex_c[i16] = zero16
        ones_c[i16] = one16
        return carry
    lax.fori_loop(0, KB // 16, _zbuf, 0)

    r0 = s * ROWS_PER_TILE
    n_full = ROWS_PER_TILE // KB
    for k in range(ROWS_PER_TILE // KB):
        pltpu.sync_copy(hrows, out_sh.at[pl.ds(r0 + k * KB, KB)])
        pltpu.sync_copy(ex_c, den_sh.at[pl.ds(r0 + k * KB, KB)])
        pltpu.sync_copy(ex_c, deg_sh.at[pl.ds(r0 + k * KB, KB)])
        pltpu.sync_copy(ex_c, sw_sh.at[pl.ds(r0 + k * KB, KB)])
    rem = ROWS_PER_TILE - (ROWS_PER_TILE // KB) * KB
    if rem:
        rb = r0 + (ROWS_PER_TILE // KB) * KB
        pltpu.sync_copy(hrows.at[pl.ds(0, rem)], out_sh.at[pl.ds(rb, rem)])
        pltpu.sync_copy(ex_c.at[pl.ds(0, rem)], den_sh.at[pl.ds(rb, rem)])
        pltpu.sync_copy(ex_c.at[pl.ds(0, rem)], deg_sh.at[pl.ds(rb, rem)])
        pltpu.sync_copy(ex_c.at[pl.ds(0, rem)], sw_sh.at[pl.ds(rb, rem)])

    plsc.subcore_barrier()

    base = (c * NS + s) * EDGES_PER_TILE
    hrowsB = (hrows, hrows2)
    srcB = (src_c, src_c2)
    dstB = (dst_c, dst_c2)
    ewB = (ew_c, ew_c2)
    exB = (ex_c, ex_c2)
    avB = (av_c, av_c2)
    bvB = (bv_c, bv_c2)
    semLB = (semL0, semL1)
    semGB = (semG0, semG1)

    def _issue_loads(b, t):
        e0 = base + t * KB
        pltpu.async_copy(src_hbm.at[pl.ds(e0, KB)], srcB[b], semLB[b])
        pltpu.async_copy(dst_hbm.at[pl.ds(e0, KB)], dstB[b], semLB[b])
        pltpu.async_copy(ew_hbm.at[pl.ds(e0, KB)], ewB[b], semLB[b])

    def _wait_loads(b):
        pltpu.make_async_copy(src_hbm.at[pl.ds(0, KB)], srcB[b], semLB[b]).wait()
        pltpu.make_async_copy(dst_hbm.at[pl.ds(0, KB)], dstB[b], semLB[b]).wait()
        pltpu.make_async_copy(ew_hbm.at[pl.ds(0, KB)], ewB[b], semLB[b]).wait()

    def _issue_gathers(b):
        pltpu.async_copy(h_hbm.at[srcB[b]], hrowsB[b], semGB[b])
        pltpu.async_copy(asrc_hbm.at[srcB[b]], avB[b], semGB[b])
        pltpu.async_copy(adst_hbm.at[dstB[b]], bvB[b], semGB[b])

    def _wait_gathers(b):
        pltpu.make_async_copy(h_hbm.at[pl.ds(0, KB)], hrowsB[b], semGB[b]).wait()
        pltpu.make_async_copy(asrc_hbm.at[pl.ds(0, KB)], avB[b], semGB[b]).wait()
        pltpu.make_async_copy(adst_hbm.at[pl.ds(0, KB)], bvB[b], semGB[b]).wait()

    _issue_loads(0, 0)
    _wait_loads(0)
    _issue_gathers(0)

    def _macro(i, carry):
        for b in (0, 1):
            t = 2 * i + b
            tn = lax.rem(t + 1, CHUNKS)
            nb = 1 - b
            _issue_loads(nb, tn)

            _wait_gathers(b)

            def _grp(g, cc):
                i16 = pl.ds(g * 16, 16)
                al = avB[b][i16] + bvB[b][i16] + ewB[b][i16]
                al = jnp.where(al >= 0.0, al, 0.2 * al)
                exB[b][i16] = jnp.exp(al)
                return cc
            lax.fori_loop(0, KB // 16, _grp, 0, unroll=8)

            def _scale(j, cc):
                sv = plsc.load_gather(exB[b], [jnp.full((16,), j, jnp.int32)])
                hrowsB[b][j, 0:16] = hrowsB[b][j, 0:16] * sv
                hrowsB[b][j, 16:32] = hrowsB[b][j, 16:32] * sv
                return cc
            lax.fori_loop(0, KB, _scale, 0, unroll=8)

            _wait_loads(nb)
            _issue_gathers(nb)

            pltpu.sync_copy(hrowsB[b], out_sh.at[dstB[b]], add=True)
            pltpu.sync_copy(exB[b], den_sh.at[dstB[b]], add=True)
            pltpu.sync_copy(ones_c, deg_sh.at[dstB[b]], add=True)
            pltpu.sync_copy(ewB[b], sw_sh.at[dstB[b]], add=True)
        return carry

    lax.fori_loop(0, CHUNKS // 2, _macro, 0)

    _wait_gathers(0)

    plsc.subcore_barrier()

    w0 = c * N_PAD + r0
    pltpu.sync_copy(out_sh.at[pl.ds(r0, ROWS_PER_TILE)],
                    out_hbm.at[pl.ds(w0, ROWS_PER_TILE)])
    pltpu.sync_copy(den_sh.at[pl.ds(r0, ROWS_PER_TILE)],
                    den_hbm.at[pl.ds(w0, ROWS_PER_TILE)])
    pltpu.sync_copy(deg_sh.at[pl.ds(r0, ROWS_PER_TILE)],
                    deg_hbm.at[pl.ds(w0, ROWS_PER_TILE)])
    pltpu.sync_copy(sw_sh.at[pl.ds(r0, ROWS_PER_TILE)],
                    sw_hbm.at[pl.ds(w0, ROWS_PER_TILE)])


def _edge_pass(srcp, dstp, ewp, asrc, adst, h):
    mesh = plsc.VectorSubcoreMesh(
        core_axis_name="c", subcore_axis_name="s",
        num_cores=NC, num_subcores=NS)
    return pl.kernel(
        _edge_body,
        out_type=[
            jax.ShapeDtypeStruct((NC * N_PAD, 32), jnp.float32),
            jax.ShapeDtypeStruct((NC * N_PAD,), jnp.float32),
            jax.ShapeDtypeStruct((NC * N_PAD,), jnp.float32),
            jax.ShapeDtypeStruct((NC * N_PAD,), jnp.float32),
        ],
        mesh=mesh,
        scratch_types=[
            pltpu.VMEM((KB, 32), jnp.float32),
            pltpu.VMEM((KB,), jnp.int32),
            pltpu.VMEM((KB,), jnp.int32),
            pltpu.VMEM((KB,), jnp.float32),
            pltpu.VMEM((KB,), jnp.float32),
            pltpu.VMEM((KB,), jnp.float32),
            pltpu.VMEM((KB,), jnp.float32),
            pltpu.VMEM((KB,), jnp.float32),
            pltpu.VMEM((KB, 32), jnp.float32),
            pltpu.VMEM((KB,), jnp.int32),
            pltpu.VMEM((KB,), jnp.int32),
            pltpu.VMEM((KB,), jnp.float32),
            pltpu.VMEM((KB,), jnp.float32),
            pltpu.VMEM((KB,), jnp.float32),
            pltpu.VMEM((KB,), jnp.float32),
            pltpu.VMEM_SHARED((N_PAD, 32), jnp.float32),
            pltpu.VMEM_SHARED((N_PAD,), jnp.float32),
            pltpu.VMEM_SHARED((N_PAD,), jnp.float32),
            pltpu.VMEM_SHARED((N_PAD,), jnp.float32),
            pltpu.SemaphoreType.DMA,
            pltpu.SemaphoreType.DMA,
            pltpu.SemaphoreType.DMA,
            pltpu.SemaphoreType.DMA,
        ],
        compiler_params=pltpu.CompilerParams(needs_layout_passes=False, use_tc_tiling_on_sc=False),
    )(srcp, dstp, ewp, asrc, adst, h)


def _epilogue_body(op_ref, s_ref, h_ref, m_ref, v_ref, b_ref, y_ref):
    S = s_ref[...]
    den = S[:, 0:1] + S[:, 1:2]
    deg = S[:, 2:3] + S[:, 3:4]
    sw = S[:, 4:5] + S[:, 5:6]
    al = S[:, 6:7] + S[:, 7:8] + sw / jnp.maximum(deg, 1.0)
    al = jnp.where(al >= 0.0, al, 0.2 * al)
    exs = jnp.exp(al)
    h = h_ref[...]
    out = (op_ref[0] + op_ref[1] + exs * h) / (den + exs + 1e-16)
    xr = jnp.maximum(out + v_ref[0:1, :], 0.0)
    bias = jnp.sum(xr * v_ref[1:2, :], axis=1, keepdims=True) + v_ref[3:4, 0:1]
    b_ref[...] = bias * m_ref[...]
    y_ref[...] = (jnp.sum(xr * v_ref[2:3, :], axis=1, keepdims=True)
                  + v_ref[3:4, 1:2])


def _epilogue(out_p, scal, h, maskp, vecs):
    nb = N_PAD // BLK
    return pl.pallas_call(
        _epilogue_body,
        grid=(nb,),
        in_specs=[
            pl.BlockSpec((2, BLK, 32), lambda i: (0, i, 0)),
            pl.BlockSpec((BLK, 8), lambda i: (i, 0)),
            pl.BlockSpec((BLK, 32), lambda i: (i, 0)),
            pl.BlockSpec((BLK, 1), lambda i: (i, 0)),
            pl.BlockSpec((4, 32), lambda i: (0, 0)),
        ],
        out_specs=[
            pl.BlockSpec((BLK, 1), lambda i: (i, 0)),
            pl.BlockSpec((BLK, 1), lambda i: (i, 0)),
        ],
        out_shape=[
            jax.ShapeDtypeStruct((N_PAD, 1), jnp.float32),
            jax.ShapeDtypeStruct((N_PAD, 1), jnp.float32),
        ],
    )(out_p, scal, h, maskp, vecs)


def _weights_body(src_hbm, dst_hbm, yp_hbm, w_hbm, yp_v, src_c, dst_c, w_c):
    c = lax.axis_index("c")
    s = lax.axis_index("s")
    pltpu.sync_copy(yp_hbm, yp_v)
    base = (c * NS + s) * EDGES_PER_TILE

    def _chunk(t, carry):
        e0 = base + t * KD
        pltpu.sync_copy(src_hbm.at[pl.ds(e0, KD)], src_c)
        pltpu.sync_copy(dst_hbm.at[pl.ds(e0, KD)], dst_c)

        def _grp(g, cc):
            i16 = pl.ds(g * 16, 16)
            a = plsc.load_gather(yp_v, [src_c[i16]])
            b = plsc.load_gather(yp_v, [dst_c[i16]])
            w_c[i16] = 0.5 * (a + b)
            return cc
        lax.fori_loop(0, KD // 16, _grp, 0, unroll=8)

        pltpu.sync_copy(w_c, w_hbm.at[pl.ds(e0, KD)])
        return carry

    lax.fori_loop(0, CHUNKS_D, _chunk, 0)


def _edge_weights(srcp, dstp, yp):
    mesh = plsc.VectorSubcoreMesh(
        core_axis_name="c", subcore_axis_name="s",
        num_cores=NC, num_subcores=NS)
    return pl.kernel(
        _weights_body,
        out_type=jax.ShapeDtypeStruct((E_PAD,), jnp.float32),
        mesh=mesh,
        scratch_types=[
            pltpu.VMEM((N_PAD,), jnp.float32),
            pltpu.VMEM((KD,), jnp.int32),
            pltpu.VMEM((KD,), jnp.int32),
            pltpu.VMEM((KD,), jnp.float32),
        ],
        compiler_params=pltpu.CompilerParams(needs_layout_passes=False, use_tc_tiling_on_sc=False),
    )(srcp, dstp, yp)


def kernel(x, edge_index, edge_weight, input_mask, W_src, att_src, att_dst,
           W_edge, att_edge, bias_gat, W_B, b_B, W_W, b_W):
    n, d_in = x.shape
    e = edge_index.shape[1]

    src = edge_index[0]
    dst = edge_index[1]

    pad_e = E_PAD - e
    pad_idx = jnp.full((pad_e,), n, jnp.int32)
    srcp = jnp.concatenate([src, pad_idx])
    dstp = jnp.concatenate([dst, pad_idx])

    c1 = jnp.dot(W_edge[0], att_edge)
    ewp = jnp.concatenate([edge_weight[:, 0] * c1,
                           jnp.zeros((pad_e,), jnp.float32)])

    v_src = W_src @ att_src
    v_dst = W_src @ att_dst
    w_aug = jnp.concatenate(
        [W_src, v_src[:, None], v_dst[:, None]], axis=1)
    w_aug = jnp.pad(w_aug, ((0, 0), (0, 128 - 34)))

    h_aug = _project(x, w_aug)
    h = jnp.pad(h_aug[:, :32], ((0, N_PAD - n), (0, 0)))
    asrc = jnp.pad(h_aug[:, 32], (0, N_PAD - n))
    adst = jnp.pad(h_aug[:, 33], (0, N_PAD - n))

    out_f, den_f, deg_f, sw_f = _edge_pass(srcp, dstp, ewp, asrc, adst, h)
    out_p = out_f.reshape(NC, N_PAD, 32)
    den_p = den_f.reshape(NC, N_PAD)
    deg_p = deg_f.reshape(NC, N_PAD)
    sw_p = sw_f.reshape(NC, N_PAD)

    scal = jnp.stack(
        [den_p[0], den_p[1], deg_p[0], deg_p[1], sw_p[0], sw_p[1],
         asrc, adst], axis=1)
    maskp = jnp.pad(input_mask, ((0, N_PAD - n), (0, 0)))
    vecs = jnp.stack(
        [bias_gat, W_B[:, 0], W_W[:, 0],
         jnp.concatenate([b_B, b_W, jnp.zeros((30,), jnp.float32)])])

    biases_full, yp_full = _epilogue(out_p, scal, h, maskp, vecs)

    w_pad = _edge_weights(srcp, dstp, yp_full[:, 0])

    return (w_pad[:e, None], biases_full[:n])


# speedup vs baseline: 29.7289x; 1.0015x over previous
import jax
import jax.numpy as jnp
from jax import lax
from jax.experimental import pallas as pl
from jax.experimental.pallas import tpu as pltpu
from jax.experimental.pallas import tpu_sc as plsc

NC = 2
NS = 16
NW = NC * NS

N_PAD = 50176
ROWS_PER_TILE = N_PAD // NS
E_PAD = 819200
EDGES_PER_TILE = E_PAD // NW
KB = 128
CHUNKS = EDGES_PER_TILE // KB
KD = 1024
CHUNKS_D = EDGES_PER_TILE // KD
BLK = 512


def _mm_body(x_ref, w_ref, o_ref):
    o_ref[...] = lax.dot_general(
        x_ref[...], w_ref[...], (((1,), (0,)), ((), ())),
        precision=lax.Precision.HIGHEST)


def _project(x, w_aug):
    n, d_in = x.shape
    blk = 400
    return pl.pallas_call(
        _mm_body,
        grid=(n // blk,),
        in_specs=[
            pl.BlockSpec((blk, d_in), lambda i: (i, 0)),
            pl.BlockSpec((d_in, 128), lambda i: (0, 0)),
        ],
        out_specs=pl.BlockSpec((blk, 128), lambda i: (i, 0)),
        out_shape=jax.ShapeDtypeStruct((n, 128), jnp.float32),
    )(x, w_aug)


def _edge_body(src_hbm, dst_hbm, ew_hbm, asrc_hbm, adst_hbm, h_hbm,
               out_hbm, den_hbm, deg_hbm, sw_hbm,
               hrows, src_c, dst_c, ew_c, ex_c, ones_c, av_c, bv_c,
               hrows2, src_c2, dst_c2, ew_c2, ex_c2, av_c2, bv_c2,
               out_sh, den_sh, deg_sh, sw_sh,
               semL0, semL1, semG0, semG1):
    c = lax.axis_index("c")
    s = lax.axis_index("s")

    zero16 = jnp.zeros((16,), jnp.float32)
    one16 = jnp.ones((16,), jnp.float32)

    def _zrow(j, carry):
        hrows[j, 0:16] = zero16
        hrows[j, 16:32] = zero16
        return carry
    lax.fori_loop(0, KB, _zrow, 0)

    def _zbuf(g, carry):
        i16 = pl.ds(g * 16, 16)
        ex_c[i16] = zero16
        ones_c[i16] = one16
        return carry
    lax.fori_loop(0, KB // 16, _zbuf, 0)

    r0 = s * ROWS_PER_TILE
    for k in range(ROWS_PER_TILE // KB):
        pltpu.sync_copy(hrows, out_sh.at[pl.ds(r0 + k * KB, KB)])
        pltpu.sync_copy(ex_c, den_sh.at[pl.ds(r0 + k * KB, KB)])
        pltpu.sync_copy(ex_c, deg_sh.at[pl.ds(r0 + k * KB, KB)])
        pltpu.sync_copy(ex_c, sw_sh.at[pl.ds(r0 + k * KB, KB)])
    rem = ROWS_PER_TILE - (ROWS_PER_TILE // KB) * KB
    if rem:
        rb = r0 + (ROWS_PER_TILE // KB) * KB
        pltpu.sync_copy(hrows.at[pl.ds(0, rem)], out_sh.at[pl.ds(rb, rem)])
        pltpu.sync_copy(ex_c.at[pl.ds(0, rem)], den_sh.at[pl.ds(rb, rem)])
        pltpu.sync_copy(ex_c.at[pl.ds(0, rem)], deg_sh.at[pl.ds(rb, rem)])
        pltpu.sync_copy(ex_c.at[pl.ds(0, rem)], sw_sh.at[pl.ds(rb, rem)])

    plsc.subcore_barrier()

    base = (c * NS + s) * EDGES_PER_TILE
    hrowsB = (hrows, hrows2)
    srcB = (src_c, src_c2)
    dstB = (dst_c, dst_c2)
    ewB = (ew_c, ew_c2)
    exB = (ex_c, ex_c2)
    avB = (av_c, av_c2)
    bvB = (bv_c, bv_c2)
    semLB = (semL0, semL1)
    semGB = (semG0, semG1)

    def _issue_loads(b, t):
        e0 = base + t * KB
        pltpu.async_copy(src_hbm.at[pl.ds(e0, KB)], srcB[b], semLB[b])
        pltpu.async_copy(dst_hbm.at[pl.ds(e0, KB)], dstB[b], semLB[b])
        pltpu.async_copy(ew_hbm.at[pl.ds(e0, KB)], ewB[b], semLB[b])

    def _wait_loads(b):
        pltpu.make_async_copy(src_hbm.at[pl.ds(0, KB)], srcB[b], semLB[b]).wait()
        pltpu.make_async_copy(dst_hbm.at[pl.ds(0, KB)], dstB[b], semLB[b]).wait()
        pltpu.make_async_copy(ew_hbm.at[pl.ds(0, KB)], ewB[b], semLB[b]).wait()

    def _issue_gathers(b):
        pltpu.async_copy(h_hbm.at[srcB[b]], hrowsB[b], semGB[b])
        pltpu.async_copy(asrc_hbm.at[srcB[b]], avB[b], semGB[b])
        pltpu.async_copy(adst_hbm.at[dstB[b]], bvB[b], semGB[b])

    def _wait_gathers(b):
        pltpu.make_async_copy(h_hbm.at[pl.ds(0, KB)], hrowsB[b], semGB[b]).wait()
        pltpu.make_async_copy(asrc_hbm.at[pl.ds(0, KB)], avB[b], semGB[b]).wait()
        pltpu.make_async_copy(adst_hbm.at[pl.ds(0, KB)], bvB[b], semGB[b]).wait()

    _issue_loads(0, 0)
    _wait_loads(0)
    _issue_gathers(0)

    def _macro(i, carry):
        for b in (0, 1):
            t = 2 * i + b
            tn = lax.rem(t + 1, CHUNKS)
            nb = 1 - b
            _issue_loads(nb, tn)

            _wait_gathers(b)

            def _grp(g, cc):
                i16 = pl.ds(g * 16, 16)
                al = avB[b][i16] + bvB[b][i16] + ewB[b][i16]
                al = jnp.where(al >= 0.0, al, 0.2 * al)
                exB[b][i16] = jnp.exp(al)
                return cc
            lax.fori_loop(0, KB // 16, _grp, 0, unroll=8)

            def _scale(j, cc):
                sv = plsc.load_gather(exB[b], [jnp.full((16,), j, jnp.int32)])
                hrowsB[b][j, 0:16] = hrowsB[b][j, 0:16] * sv
                hrowsB[b][j, 16:32] = hrowsB[b][j, 16:32] * sv
                return cc
            lax.fori_loop(0, KB, _scale, 0, unroll=8)

            _wait_loads(nb)
            _issue_gathers(nb)

            pltpu.sync_copy(hrowsB[b], out_sh.at[dstB[b]], add=True)
            pltpu.sync_copy(exB[b], den_sh.at[dstB[b]], add=True)
            pltpu.sync_copy(ones_c, deg_sh.at[dstB[b]], add=True)
            pltpu.sync_copy(ewB[b], sw_sh.at[dstB[b]], add=True)
        return carry

    lax.fori_loop(0, CHUNKS // 2, _macro, 0)

    _wait_gathers(0)

    plsc.subcore_barrier()

    w0 = c * N_PAD + r0
    pltpu.sync_copy(out_sh.at[pl.ds(r0, ROWS_PER_TILE)],
                    out_hbm.at[pl.ds(w0, ROWS_PER_TILE)])
    pltpu.sync_copy(den_sh.at[pl.ds(r0, ROWS_PER_TILE)],
                    den_hbm.at[pl.ds(w0, ROWS_PER_TILE)])
    pltpu.sync_copy(deg_sh.at[pl.ds(r0, ROWS_PER_TILE)],
                    deg_hbm.at[pl.ds(w0, ROWS_PER_TILE)])
    pltpu.sync_copy(sw_sh.at[pl.ds(r0, ROWS_PER_TILE)],
                    sw_hbm.at[pl.ds(w0, ROWS_PER_TILE)])


def _edge_pass(srcp, dstp, ewp, asrc, adst, h):
    mesh = plsc.VectorSubcoreMesh(
        core_axis_name="c", subcore_axis_name="s",
        num_cores=NC, num_subcores=NS)
    return pl.kernel(
        _edge_body,
        out_type=[
            jax.ShapeDtypeStruct((NC * N_PAD, 32), jnp.float32),
            jax.ShapeDtypeStruct((NC * N_PAD,), jnp.float32),
            jax.ShapeDtypeStruct((NC * N_PAD,), jnp.float32),
            jax.ShapeDtypeStruct((NC * N_PAD,), jnp.float32),
        ],
        mesh=mesh,
        scratch_types=[
            pltpu.VMEM((KB, 32), jnp.float32),
            pltpu.VMEM((KB,), jnp.int32),
            pltpu.VMEM((KB,), jnp.int32),
            pltpu.VMEM((KB,), jnp.float32),
            pltpu.VMEM((KB,), jnp.float32),
            pltpu.VMEM((KB,), jnp.float32),
            pltpu.VMEM((KB,), jnp.float32),
            pltpu.VMEM((KB,), jnp.float32),
            pltpu.VMEM((KB, 32), jnp.float32),
            pltpu.VMEM((KB,), jnp.int32),
            pltpu.VMEM((KB,), jnp.int32),
            pltpu.VMEM((KB,), jnp.float32),
            pltpu.VMEM((KB,), jnp.float32),
            pltpu.VMEM((KB,), jnp.float32),
            pltpu.VMEM((KB,), jnp.float32),
            pltpu.VMEM_SHARED((N_PAD, 32), jnp.float32),
            pltpu.VMEM_SHARED((N_PAD,), jnp.float32),
            pltpu.VMEM_SHARED((N_PAD,), jnp.float32),
            pltpu.VMEM_SHARED((N_PAD,), jnp.float32),
            pltpu.SemaphoreType.DMA,
            pltpu.SemaphoreType.DMA,
            pltpu.SemaphoreType.DMA,
            pltpu.SemaphoreType.DMA,
        ],
        compiler_params=pltpu.CompilerParams(needs_layout_passes=False, use_tc_tiling_on_sc=False),
    )(srcp, dstp, ewp, asrc, adst, h)


def _epilogue_body(op_ref, s_ref, h_ref, m_ref, v_ref, b_ref, y_ref):
    S = s_ref[...]
    den = S[:, 0:1] + S[:, 1:2]
    deg = S[:, 2:3] + S[:, 3:4]
    sw = S[:, 4:5] + S[:, 5:6]
    al = S[:, 6:7] + S[:, 7:8] + sw / jnp.maximum(deg, 1.0)
    al = jnp.where(al >= 0.0, al, 0.2 * al)
    exs = jnp.exp(al)
    h = h_ref[...]
    out = (op_ref[0] + op_ref[1] + exs * h) / (den + exs + 1e-16)
    xr = jnp.maximum(out + v_ref[0:1, :], 0.0)
    bias = jnp.sum(xr * v_ref[1:2, :], axis=1, keepdims=True) + v_ref[3:4, 0:1]
    b_ref[...] = bias * m_ref[...]
    y_ref[...] = (jnp.sum(xr * v_ref[2:3, :], axis=1, keepdims=True)
                  + v_ref[3:4, 1:2])


def _epilogue(out_p, scal, h, maskp, vecs):
    nb = N_PAD // BLK
    return pl.pallas_call(
        _epilogue_body,
        grid=(nb,),
        in_specs=[
            pl.BlockSpec((2, BLK, 32), lambda i: (0, i, 0)),
            pl.BlockSpec((BLK, 8), lambda i: (i, 0)),
            pl.BlockSpec((BLK, 32), lambda i: (i, 0)),
            pl.BlockSpec((BLK, 1), lambda i: (i, 0)),
            pl.BlockSpec((4, 32), lambda i: (0, 0)),
        ],
        out_specs=[
            pl.BlockSpec((BLK, 1), lambda i: (i, 0)),
            pl.BlockSpec((BLK, 1), lambda i: (i, 0)),
        ],
        out_shape=[
            jax.ShapeDtypeStruct((N_PAD, 1), jnp.float32),
            jax.ShapeDtypeStruct((N_PAD, 1), jnp.float32),
        ],
    )(out_p, scal, h, maskp, vecs)


def _weights_body(src_hbm, dst_hbm, yp_hbm, w_hbm, yp_v, src_c, dst_c, w_c):
    c = lax.axis_index("c")
    s = lax.axis_index("s")
    pltpu.sync_copy(yp_hbm, yp_v)
    base = (c * NS + s) * EDGES_PER_TILE

    def _chunk(t, carry):
        e0 = base + t * KD
        pltpu.sync_copy(src_hbm.at[pl.ds(e0, KD)], src_c)
        pltpu.sync_copy(dst_hbm.at[pl.ds(e0, KD)], dst_c)

        def _grp(g, cc):
            i16 = pl.ds(g * 16, 16)
            a = plsc.load_gather(yp_v, [src_c[i16]])
            b = plsc.load_gather(yp_v, [dst_c[i16]])
            w_c[i16] = 0.5 * (a + b)
            return cc
        lax.fori_loop(0, KD // 16, _grp, 0, unroll=8)

        pltpu.sync_copy(w_c, w_hbm.at[pl.ds(e0, KD)])
        return carry

    lax.fori_loop(0, CHUNKS_D, _chunk, 0)


def _edge_weights(srcp, dstp, yp):
    mesh = plsc.VectorSubcoreMesh(
        core_axis_name="c", subcore_axis_name="s",
        num_cores=NC, num_subcores=NS)
    return pl.kernel(
        _weights_body,
        out_type=jax.ShapeDtypeStruct((E_PAD,), jnp.float32),
        mesh=mesh,
        scratch_types=[
            pltpu.VMEM((N_PAD,), jnp.float32),
            pltpu.VMEM((KD,), jnp.int32),
            pltpu.VMEM((KD,), jnp.int32),
            pltpu.VMEM((KD,), jnp.float32),
        ],
        compiler_params=pltpu.CompilerParams(needs_layout_passes=False, use_tc_tiling_on_sc=False),
    )(srcp, dstp, yp)


def kernel(x, edge_index, edge_weight, input_mask, W_src, att_src, att_dst,
           W_edge, att_edge, bias_gat, W_B, b_B, W_W, b_W):
    n, d_in = x.shape
    e = edge_index.shape[1]

    src = edge_index[0]
    dst = edge_index[1]

    pad_e = E_PAD - e
    pad_idx = jnp.full((pad_e,), n, jnp.int32)
    srcp = jnp.concatenate([src, pad_idx])
    dstp = jnp.concatenate([dst, pad_idx])

    c1 = jnp.dot(W_edge[0], att_edge)
    ewp = jnp.concatenate([edge_weight[:, 0] * c1,
                           jnp.zeros((pad_e,), jnp.float32)])

    v_src = W_src @ att_src
    v_dst = W_src @ att_dst
    w_aug = jnp.concatenate(
        [W_src, v_src[:, None], v_dst[:, None]], axis=1)
    w_aug = jnp.pad(w_aug, ((0, 0), (0, 128 - 34)))

    h_aug = _project(x, w_aug)
    h = jnp.pad(h_aug[:, :32], ((0, N_PAD - n), (0, 0)))
    asrc = jnp.pad(h_aug[:, 32], (0, N_PAD - n))
    adst = jnp.pad(h_aug[:, 33], (0, N_PAD - n))

    out_f, den_f, deg_f, sw_f = _edge_pass(srcp, dstp, ewp, asrc, adst, h)
    out_p = out_f.reshape(NC, N_PAD, 32)
    den_p = den_f.reshape(NC, N_PAD)
    deg_p = deg_f.reshape(NC, N_PAD)
    sw_p = sw_f.reshape(NC, N_PAD)

    scal = jnp.stack(
        [den_p[0], den_p[1], deg_p[0], deg_p[1], sw_p[0], sw_p[1],
         asrc, adst], axis=1)
    maskp = jnp.pad(input_mask, ((0, N_PAD - n), (0, 0)))
    vecs = jnp.stack(
        [bias_gat, W_B[:, 0], W_W[:, 0],
         jnp.concatenate([b_B, b_W, jnp.zeros((30,), jnp.float32)])])

    biases_full, yp_full = _epilogue(out_p, scal, h, maskp, vecs)

    w_pad = _edge_weights(srcp, dstp, yp_full[:, 0])

    return (w_pad[:e, None], biases_full[:n])
